# baseline (speedup 1.0000x reference)
import functools

import jax
import jax.numpy as jnp
from jax import lax
from jax.experimental import pallas as pl
from jax.experimental.pallas import tpu as pltpu
from jax.experimental.pallas import tpu_sc as plsc

NUM_EMB = 100000
EMB_DIM = 64
BASE_DIM = 128
BATCH = 4096
HIST = 50

_INFO = plsc.get_sparse_core_info()
_NC, _NS = _INFO.num_cores, _INFO.num_subcores
_NW = _NC * _NS
_PASSES = EMB_DIM // _NW
_LANES = 16
_NIDX = BATCH * HIST
_NCHUNK = 32
_CIDX = _NIDX // _NCHUNK
_CBAGS = _CIDX // HIST
_GROUPS = _CBAGS // _LANES


def _sc_bagsum_fm(tableT, idx):
    mesh = plsc.VectorSubcoreMesh(core_axis_name="c", subcore_axis_name="s")

    @functools.partial(
        pl.kernel,
        mesh=mesh,
        out_type=jax.ShapeDtypeStruct((EMB_DIM, BATCH), jnp.float32),
        compiler_params=pltpu.CompilerParams(use_tc_tiling_on_sc=False,
                                             needs_layout_passes=False),
        scratch_types=[
            pltpu.VMEM((NUM_EMB,), jnp.float32),
            pltpu.VMEM((_CIDX,), jnp.int32),
            pltpu.VMEM((_CIDX,), jnp.int32),
            pltpu.VMEM((BATCH,), jnp.float32),
            pltpu.SemaphoreType.DMA,
            pltpu.SemaphoreType.DMA,
            pltpu.SemaphoreType.DMA,
        ],
    )
    def sc_kernel(tab_hbm, idx_hbm, out_hbm, trow, ib0, ib1, orow,
                  tsem, s0, s1):
        wid = lax.axis_index("s") * _NC + lax.axis_index("c")
        ibufs = (ib0, ib1)
        isems = (s0, s1)
        lane50 = lax.iota(jnp.int32, 16) * HIST
        zero = jnp.zeros((_LANES,), jnp.float32)

        def igather(c, p):
            pltpu.async_copy(idx_hbm.at[pl.ds(c * _CIDX, _CIDX)],
                             ibufs[p], isems[p])

        def idrain(p):
            pltpu.make_async_copy(idx_hbm.at[pl.ds(0, _CIDX)],
                                  ibufs[p], isems[p]).wait()

        def chunk(c, p):
            ibuf = ibufs[p]

            def group(g, _):
                av = lane50 + g * (HIST * _LANES)
                accs = [zero, zero, zero, zero]
                for h in range(HIST):
                    iv = plsc.load_gather(ibuf, [av + h])
                    tv = plsc.load_gather(trow, [iv])
                    accs[h % 4] = accs[h % 4] + tv
                orow[pl.ds(c * _CBAGS + g * _LANES, _LANES)] = (
                    (accs[0] + accs[1]) + (accs[2] + accs[3]))
                return 0

            lax.fori_loop(0, _GROUPS, group, 0)

        for p in range(_PASSES):
            f = wid * _PASSES + p
            pltpu.async_copy(tab_hbm.at[f], trow, tsem)
            igather(0, 0)
            igather(1, 1)
            pltpu.make_async_copy(tab_hbm.at[f], trow, tsem).wait()

            def cbody(it, _):
                c0 = it * 2
                idrain(0)
                chunk(c0, 0)

                @pl.when(c0 + 2 < _NCHUNK)
                def _():
                    igather(c0 + 2, 0)

                idrain(1)
                chunk(c0 + 1, 1)

                @pl.when(c0 + 3 < _NCHUNK)
                def _():
                    igather(c0 + 3, 1)

                return 0

            lax.fori_loop(0, _NCHUNK // 2, cbody, 0)
            pltpu.sync_copy(orow, out_hbm.at[f])

    return sc_kernel


def _proj_body(x_ref, w_ref, o_ref):
    o_ref[...] = lax.dot_general(
        x_ref[...], w_ref[...],
        (((0,), (1,)), ((), ())),
        preferred_element_type=jnp.float32,
        precision=lax.Precision.HIGHEST,
    )


def _tc_proj(embT, W):
    blk = 1024
    return pl.pallas_call(
        _proj_body,
        grid=(BATCH // blk,),
        in_specs=[
            pl.BlockSpec((EMB_DIM, blk), lambda i: (0, i)),
            pl.BlockSpec((BASE_DIM, EMB_DIM), lambda i: (0, 0)),
        ],
        out_specs=pl.BlockSpec((blk, BASE_DIM), lambda i: (i, 0)),
        out_shape=jax.ShapeDtypeStruct((BATCH, BASE_DIM), jnp.float32),
    )(embT, W)


def kernel(input, table, W):
    idx = input.astype(jnp.int32).reshape(-1)
    tableT = jnp.swapaxes(table, 0, 1)
    embT = _sc_bagsum_fm(tableT, idx)(tableT, idx)
    return _tc_proj(embT, W)


# speedup vs baseline: 1.3143x; 1.3143x over previous
import functools

import jax
import jax.numpy as jnp
from jax import lax
from jax.experimental import pallas as pl
from jax.experimental.pallas import tpu as pltpu
from jax.experimental.pallas import tpu_sc as plsc

NUM_EMB = 100000
EMB_DIM = 64
BASE_DIM = 128
BATCH = 4096
HIST = 50

_INFO = plsc.get_sparse_core_info()
_NC, _NS = _INFO.num_cores, _INFO.num_subcores
_NW = _NC * _NS
_BAGS_PER_W = BATCH // _NW
_IDX_PER_W = _BAGS_PER_W * HIST
_CHUNK_BAGS = 16
_CHUNK_ROWS = _CHUNK_BAGS * HIST
_LANES = 16
_VPR = EMB_DIM // _LANES


def _sc_bagsum(table, idx):
    mesh = plsc.VectorSubcoreMesh(core_axis_name="c", subcore_axis_name="s")

    @functools.partial(
        pl.kernel,
        mesh=mesh,
        out_type=jax.ShapeDtypeStruct((BATCH, EMB_DIM), jnp.float32),
        compiler_params=pltpu.CompilerParams(use_tc_tiling_on_sc=False),
        scratch_types=[
            pltpu.VMEM((_IDX_PER_W,), jnp.int32),
            pltpu.VMEM((_CHUNK_ROWS, EMB_DIM), jnp.float32),
            pltpu.VMEM((_CHUNK_ROWS, EMB_DIM), jnp.float32),
            pltpu.VMEM((_BAGS_PER_W, EMB_DIM), jnp.float32),
            pltpu.SemaphoreType.DMA,
            pltpu.SemaphoreType.DMA,
        ],
    )
    def sc_kernel(table_hbm, idx_hbm, out_hbm, idx_v, rows0, rows1, out_v,
                  sem0, sem1):
        wid = lax.axis_index("s") * _NC + lax.axis_index("c")
        base = wid * _BAGS_PER_W
        pltpu.sync_copy(idx_hbm.at[pl.ds(wid * _IDX_PER_W, _IDX_PER_W)], idx_v)

        rows = (rows0, rows1)
        sems = (sem0, sem1)
        nbuf = 2

        def gather(c, p):
            off = c * _CHUNK_ROWS
            for g0 in range(0, _CHUNK_ROWS, 128):
                glen = min(128, _CHUNK_ROWS - g0)
                pltpu.async_copy(
                    table_hbm.at[idx_v.at[pl.ds(off + g0, glen)]],
                    rows[p].at[pl.ds(g0, glen)],
                    sems[p])

        def drain(p):
            for g0 in range(0, _CHUNK_ROWS, 128):
                glen = min(128, _CHUNK_ROWS - g0)
                pltpu.make_async_copy(
                    table_hbm.at[idx_v.at[pl.ds(g0, glen)]],
                    rows[p].at[pl.ds(g0, glen)],
                    sems[p]).wait()

        def accumulate(c, p):
            rbuf = rows[p]

            def bag_body(j, _):
                r0 = j * HIST
                acc0 = [rbuf[r0, pl.ds(i * _LANES, _LANES)]
                        for i in range(_VPR)]
                acc1 = [rbuf[r0 + 1, pl.ds(i * _LANES, _LANES)]
                        for i in range(_VPR)]
                for h in range(2, HIST, 2):
                    for i in range(_VPR):
                        acc0[i] = acc0[i] + rbuf[r0 + h,
                                                 pl.ds(i * _LANES, _LANES)]
                    for i in range(_VPR):
                        acc1[i] = acc1[i] + rbuf[r0 + h + 1,
                                                 pl.ds(i * _LANES, _LANES)]
                b = c * _CHUNK_BAGS + j
                for i in range(_VPR):
                    out_v[b, pl.ds(i * _LANES, _LANES)] = acc0[i] + acc1[i]
                return 0

            lax.fori_loop(0, _CHUNK_BAGS, bag_body, 0)

        nchunks = _BAGS_PER_W // _CHUNK_BAGS
        for p in range(nbuf):
            gather(p, p)

        def body(it, _):
            c0 = it * nbuf
            for p in range(nbuf):
                drain(p)
                accumulate(c0 + p, p)

                @pl.when(c0 + p + nbuf < nchunks)
                def _():
                    gather(c0 + p + nbuf, p)

            return 0

        lax.fori_loop(0, nchunks // nbuf, body, 0)
        pltpu.sync_copy(out_v, out_hbm.at[pl.ds(base, _BAGS_PER_W)])

    return sc_kernel


def _proj_body(x_ref, w_ref, o_ref):
    o_ref[...] = lax.dot_general(
        x_ref[...], w_ref[...],
        (((1,), (1,)), ((), ())),
        preferred_element_type=jnp.float32,
        precision=lax.Precision.HIGHEST,
    )


def _tc_proj(emb, W):
    blk = 1024
    return pl.pallas_call(
        _proj_body,
        grid=(BATCH // blk,),
        in_specs=[
            pl.BlockSpec((blk, EMB_DIM), lambda i: (i, 0)),
            pl.BlockSpec((BASE_DIM, EMB_DIM), lambda i: (0, 0)),
        ],
        out_specs=pl.BlockSpec((blk, BASE_DIM), lambda i: (i, 0)),
        out_shape=jax.ShapeDtypeStruct((BATCH, BASE_DIM), jnp.float32),
    )(emb, W)


def kernel(input, table, W):
    idx = input.astype(jnp.int32).reshape(-1)
    emb = _sc_bagsum(table, idx)(table, idx)
    return _tc_proj(emb, W)
